# baseline (device time: 199773 ns/iter reference)
import jax
import jax.numpy as jnp
from jax import lax
from jax.experimental import pallas as pl
from jax.experimental.pallas import tpu as pltpu

N_DEV = 16
B, Sq, D = 4, 256, 1024
HL, Dh = 8, 128
KVL = 2
GRP = 4
Skv = 1024
R = B * Sq
CH = R // N_DEV
SCALE = 0.08838834764831843


def _body(x_ref, wq_ref, wo_ref, k_ref, v_ref, out_ref,
          rs_buf, rs_send_sem, rs_recv_sems, ag_send_sem, ag_recv_sems,
          mid_sem):
    my = lax.axis_index("i")
    right = lax.rem(my + 1, N_DEV)
    left = lax.rem(my + N_DEV - 1, N_DEV)

    barrier = pltpu.get_barrier_semaphore()
    for nbr in (left, right):
        pl.semaphore_signal(barrier, inc=1, device_id=(nbr,),
                            device_id_type=pl.DeviceIdType.MESH)
    pl.semaphore_wait(barrier, 2)

    q = jnp.dot(x_ref[...], wq_ref[...],
                preferred_element_type=jnp.float32)
    for b in range(B):
        kb = k_ref[b]
        vb = v_ref[b]
        heads = []
        for h in range(HL):
            kv = h // GRP
            qh = q[b * Sq:(b + 1) * Sq, h * Dh:(h + 1) * Dh]
            khd = kb[:, kv * Dh:(kv + 1) * Dh]
            s = lax.dot_general(
                qh, khd, (((1,), (1,)), ((), ())),
                preferred_element_type=jnp.float32) * SCALE
            m = jnp.max(s, axis=1, keepdims=True)
            p = jnp.exp(s - m)
            l = jnp.sum(p, axis=1, keepdims=True)
            o = jnp.dot(p, vb[:, kv * Dh:(kv + 1) * Dh],
                        preferred_element_type=jnp.float32)
            heads.append(o / l)
        rowb = jnp.concatenate(heads, axis=1)
        partial_b = jnp.dot(rowb, wo_ref[...],
                            preferred_element_type=jnp.float32)
        out_ref[4 * b:4 * (b + 1)] = partial_b.reshape(4, CH, D)

    for s in range(N_DEV - 1):
        send_idx = lax.rem(my - s + N_DEV, N_DEV)
        rdma = pltpu.make_async_remote_copy(
            src_ref=out_ref.at[send_idx],
            dst_ref=rs_buf.at[s],
            send_sem=rs_send_sem,
            recv_sem=rs_recv_sems.at[s],
            device_id=(right,),
            device_id_type=pl.DeviceIdType.MESH,
        )
        rdma.start()
        rdma.wait()
        acc = lax.rem(my - 1 - s + N_DEV, N_DEV)
        out_ref[pl.ds(acc, 1)] = out_ref[pl.ds(acc, 1)] + rs_buf[s][None]

    for nbr in (left, right):
        pl.semaphore_signal(mid_sem, inc=1, device_id=(nbr,),
                            device_id_type=pl.DeviceIdType.MESH)
    pl.semaphore_wait(mid_sem, 2)

    for s in range(N_DEV - 1):
        g = lax.rem(my + 1 - s + 2 * N_DEV, N_DEV)
        rdma = pltpu.make_async_remote_copy(
            src_ref=out_ref.at[g],
            dst_ref=out_ref.at[g],
            send_sem=ag_send_sem,
            recv_sem=ag_recv_sems.at[s],
            device_id=(right,),
            device_id_type=pl.DeviceIdType.MESH,
        )
        rdma.start()
        rdma.wait()


def kernel(x, Wq, Wo, K_ext, V_ext):
    my = lax.axis_index("i")
    K_loc = lax.dynamic_slice_in_dim(K_ext, my * KVL, KVL, axis=2)
    V_loc = lax.dynamic_slice_in_dim(V_ext, my * KVL, KVL, axis=2)
    K_loc = K_loc.reshape(B, Skv, KVL * Dh)
    V_loc = V_loc.reshape(B, Skv, KVL * Dh)
    x2 = x.reshape(R, D)

    out = pl.pallas_call(
        _body,
        out_shape=jax.ShapeDtypeStruct((N_DEV, CH, D), jnp.float32),
        in_specs=[pl.BlockSpec(memory_space=pltpu.VMEM)] * 5,
        out_specs=pl.BlockSpec(memory_space=pltpu.VMEM),
        scratch_shapes=[
            pltpu.VMEM((N_DEV - 1, CH, D), jnp.float32),
            pltpu.SemaphoreType.DMA,
            pltpu.SemaphoreType.DMA((N_DEV - 1,)),
            pltpu.SemaphoreType.DMA,
            pltpu.SemaphoreType.DMA((N_DEV - 1,)),
            pltpu.SemaphoreType.REGULAR,
        ],
        compiler_params=pltpu.CompilerParams(collective_id=0),
    )(x2, Wq, Wo, K_loc, V_loc)
    return out.reshape(B, Sq, D)


# device time: 164770 ns/iter; 1.2124x vs baseline; 1.2124x over previous
import jax
import jax.numpy as jnp
from jax import lax
from jax.experimental import pallas as pl
from jax.experimental.pallas import tpu as pltpu

N_DEV = 16
B, Sq, D = 4, 256, 1024
HL, Dh = 8, 128
KVL = 2
GRP = 4
Skv = 1024
R = B * Sq
CH = R // N_DEV
SCALE = 0.08838834764831843


def _rdma(src, dst, send_sem, recv_sem, dev):
    return pltpu.make_async_remote_copy(
        src_ref=src, dst_ref=dst, send_sem=send_sem, recv_sem=recv_sem,
        device_id=(dev,), device_id_type=pl.DeviceIdType.MESH,
    )


def _body(x_ref, wq_ref, wo_ref, k_ref, v_ref, out_ref,
          rs_buf, zbufA, zbufB,
          p1_send_sem, p1_recv_sems,
          p2_send_sem, p2_recv_sems,
          p3_send_sem, p3_recv_sems):
    my = lax.axis_index("i")
    z = my // 4
    w = my % 4
    zb0 = z % 2
    zb1 = (z // 2) % 2
    wr = z * 4 + (w + 1) % 4
    wl = z * 4 + (w + 3) % 4
    pA = my + 4 - 8 * zb0
    pB = my + 8 - 16 * zb1

    barrier = pltpu.get_barrier_semaphore()
    for nbr in (wl, wr, pA, pB):
        pl.semaphore_signal(barrier, inc=1, device_id=(nbr,),
                            device_id_type=pl.DeviceIdType.MESH)
    pl.semaphore_wait(barrier, 4)

    q = jnp.dot(x_ref[...], wq_ref[...],
                preferred_element_type=jnp.float32)
    for b in range(B):
        kb = k_ref[b]
        vb = v_ref[b]
        heads = []
        for h in range(HL):
            kv = h // GRP
            qh = q[b * Sq:(b + 1) * Sq, h * Dh:(h + 1) * Dh]
            khd = kb[:, kv * Dh:(kv + 1) * Dh]
            s = lax.dot_general(
                qh, khd, (((1,), (1,)), ((), ())),
                preferred_element_type=jnp.float32) * SCALE
            m = jnp.max(s, axis=1, keepdims=True)
            p = jnp.exp(s - m)
            l = jnp.sum(p, axis=1, keepdims=True)
            o = jnp.dot(p, vb[:, kv * Dh:(kv + 1) * Dh],
                        preferred_element_type=jnp.float32)
            heads.append(o / l)
        rowb = jnp.concatenate(heads, axis=1)
        partial_b = jnp.dot(rowb, wo_ref[...],
                            preferred_element_type=jnp.float32)
        out_ref[4 * b:4 * (b + 1)] = partial_b.reshape(4, CH, D)

    for s in range(3):
        g = (w - s + 4) % 4
        r = _rdma(out_ref.at[pl.ds(4 * g, 4)], rs_buf.at[s],
                  p1_send_sem, p1_recv_sems.at[s], wr)
        r.start()
        r.wait()
        a = (w - 1 - s + 4) % 4
        out_ref[pl.ds(4 * a, 4)] = out_ref[pl.ds(4 * a, 4)] + rs_buf[s]
    G = (w + 1) % 4
    base = 4 * G
    kA = base + 2 * zb0

    sA = base + 2 * (1 - zb0)
    r = _rdma(out_ref.at[pl.ds(sA, 2)], zbufA,
              p2_send_sem, p2_recv_sems.at[0], pA)
    r.start()
    r.wait()
    out_ref[pl.ds(kA, 2)] = out_ref[pl.ds(kA, 2)] + zbufA[...]

    kb_ = kA + zb1
    sb_ = kA + (1 - zb1)
    r = _rdma(out_ref.at[pl.ds(sb_, 1)], zbufB,
              p2_send_sem, p2_recv_sems.at[1], pB)
    r.start()
    r.wait()
    out_ref[pl.ds(kb_, 1)] = out_ref[pl.ds(kb_, 1)] + zbufB[...]

    r = _rdma(out_ref.at[pl.ds(kb_, 1)], out_ref.at[pl.ds(kb_, 1)],
              p2_send_sem, p2_recv_sems.at[2], pB)
    r.start()
    r.wait()

    r = _rdma(out_ref.at[pl.ds(kA, 2)], out_ref.at[pl.ds(kA, 2)],
              p2_send_sem, p2_recv_sems.at[3], pA)
    r.start()
    r.wait()

    for s in range(3):
        g = (w + 1 - s + 4) % 4
        r = _rdma(out_ref.at[pl.ds(4 * g, 4)], out_ref.at[pl.ds(4 * g, 4)],
                  p3_send_sem, p3_recv_sems.at[s], wr)
        r.start()
        r.wait()


def kernel(x, Wq, Wo, K_ext, V_ext):
    my = lax.axis_index("i")
    K_loc = lax.dynamic_slice_in_dim(K_ext, my * KVL, KVL, axis=2)
    V_loc = lax.dynamic_slice_in_dim(V_ext, my * KVL, KVL, axis=2)
    K_loc = K_loc.reshape(B, Skv, KVL * Dh)
    V_loc = V_loc.reshape(B, Skv, KVL * Dh)
    x2 = x.reshape(R, D)

    out = pl.pallas_call(
        _body,
        out_shape=jax.ShapeDtypeStruct((N_DEV, CH, D), jnp.float32),
        in_specs=[pl.BlockSpec(memory_space=pltpu.VMEM)] * 5,
        out_specs=pl.BlockSpec(memory_space=pltpu.VMEM),
        scratch_shapes=[
            pltpu.VMEM((3, 4, CH, D), jnp.float32),
            pltpu.VMEM((2, CH, D), jnp.float32),
            pltpu.VMEM((1, CH, D), jnp.float32),
            pltpu.SemaphoreType.DMA,
            pltpu.SemaphoreType.DMA((3,)),
            pltpu.SemaphoreType.DMA,
            pltpu.SemaphoreType.DMA((4,)),
            pltpu.SemaphoreType.DMA,
            pltpu.SemaphoreType.DMA((3,)),
        ],
        compiler_params=pltpu.CompilerParams(collective_id=0),
    )(x2, Wq, Wo, K_loc, V_loc)
    return out.reshape(B, Sq, D)


# device time: 121476 ns/iter; 1.6445x vs baseline; 1.3564x over previous
import jax
import jax.numpy as jnp
from jax import lax
from jax.experimental import pallas as pl
from jax.experimental.pallas import tpu as pltpu

N_DEV = 16
B, Sq, D = 4, 256, 1024
HL, Dh = 8, 128
KVL = 2
GRP = 4
Skv = 1024
R = B * Sq
CH = R // N_DEV
SCALE = 0.08838834764831843
BF = jnp.bfloat16
F32 = jnp.float32


def _rdma(src, dst, send_sem, recv_sem, dev):
    return pltpu.make_async_remote_copy(
        src_ref=src, dst_ref=dst, send_sem=send_sem, recv_sem=recv_sem,
        device_id=(dev,), device_id_type=pl.DeviceIdType.MESH,
    )


def _body(x_ref, wq_ref, wo_ref, k_ref, v_ref, out_ref,
          st16, rs16, zA16, zB16, gb16, ga16, ag16,
          p1_send_sem, p1_recv_sems,
          p2_send_sem, p2_recv_sems,
          p3_send_sem, p3_recv_sems):
    my = lax.axis_index("i")
    z = my // 4
    w = my % 4
    zb0 = z % 2
    zb1 = (z // 2) % 2
    wr = z * 4 + (w + 1) % 4
    wl = z * 4 + (w + 3) % 4
    pA = my + 4 - 8 * zb0
    pB = my + 8 - 16 * zb1

    barrier = pltpu.get_barrier_semaphore()
    for nbr in (wl, wr, pA, pB):
        pl.semaphore_signal(barrier, inc=1, device_id=(nbr,),
                            device_id_type=pl.DeviceIdType.MESH)
    pl.semaphore_wait(barrier, 4)

    q = jnp.dot(x_ref[...], wq_ref[...],
                preferred_element_type=F32)
    for b in range(B):
        kb = k_ref[b]
        vb = v_ref[b]
        heads = []
        for h in range(HL):
            kv = h // GRP
            qh = q[b * Sq:(b + 1) * Sq, h * Dh:(h + 1) * Dh]
            khd = kb[:, kv * Dh:(kv + 1) * Dh]
            s = lax.dot_general(
                qh, khd, (((1,), (1,)), ((), ())),
                preferred_element_type=F32) * SCALE
            m = jnp.max(s, axis=1, keepdims=True)
            p = jnp.exp(s - m)
            l = jnp.sum(p, axis=1, keepdims=True)
            o = jnp.dot(p, vb[:, kv * Dh:(kv + 1) * Dh],
                        preferred_element_type=F32)
            heads.append(o / l)
        rowb = jnp.concatenate(heads, axis=1)
        partial_b = jnp.dot(rowb, wo_ref[...],
                            preferred_element_type=F32)
        out_ref[4 * b:4 * (b + 1)] = partial_b.reshape(4, CH, D)

    for s in range(3):
        g = (w - s + 4) % 4
        st16[...] = out_ref[pl.ds(4 * g, 4)].astype(BF)
        r = _rdma(st16, rs16.at[s], p1_send_sem, p1_recv_sems.at[s], wr)
        r.start()
        r.wait()
        a = (w - 1 - s + 4) % 4
        out_ref[pl.ds(4 * a, 4)] = (out_ref[pl.ds(4 * a, 4)]
                                    + rs16[s].astype(F32))
    G = (w + 1) % 4
    base = 4 * G
    kA = base + 2 * zb0

    sA = base + 2 * (1 - zb0)
    st16[pl.ds(0, 2)] = out_ref[pl.ds(sA, 2)].astype(BF)
    r = _rdma(st16.at[pl.ds(0, 2)], zA16,
              p2_send_sem, p2_recv_sems.at[0], pA)
    r.start()
    r.wait()
    out_ref[pl.ds(kA, 2)] = out_ref[pl.ds(kA, 2)] + zA16[...].astype(F32)

    kb_ = kA + zb1
    sb_ = kA + (1 - zb1)
    st16[pl.ds(0, 1)] = out_ref[pl.ds(sb_, 1)].astype(BF)
    r = _rdma(st16.at[pl.ds(0, 1)], zB16,
              p2_send_sem, p2_recv_sems.at[1], pB)
    r.start()
    r.wait()
    out_ref[pl.ds(kb_, 1)] = out_ref[pl.ds(kb_, 1)] + zB16[...].astype(F32)

    st16[pl.ds(0, 1)] = out_ref[pl.ds(kb_, 1)].astype(BF)
    r = _rdma(st16.at[pl.ds(0, 1)], gb16,
              p2_send_sem, p2_recv_sems.at[2], pB)
    r.start()
    r.wait()
    out_ref[pl.ds(sb_, 1)] = gb16[...].astype(F32)

    st16[pl.ds(zb1, 1)] = out_ref[pl.ds(kb_, 1)].astype(BF)
    st16[pl.ds(1 - zb1, 1)] = gb16[...]
    r = _rdma(st16.at[pl.ds(0, 2)], ga16,
              p2_send_sem, p2_recv_sems.at[3], pA)
    r.start()
    r.wait()
    oth = base + 2 * (1 - zb0)
    out_ref[pl.ds(oth, 2)] = ga16[...].astype(F32)

    st16[pl.ds(2 * zb0 + zb1, 1)] = out_ref[pl.ds(kb_, 1)].astype(BF)
    st16[pl.ds(2 * zb0 + 1 - zb1, 1)] = gb16[...]
    st16[pl.ds(2 * (1 - zb0), 2)] = ga16[...]
    for s in range(3):
        src = st16 if s == 0 else ag16.at[s - 1]
        r = _rdma(src, ag16.at[s], p3_send_sem, p3_recv_sems.at[s], wr)
        r.start()
        r.wait()
        gr = (w - s + 4) % 4
        out_ref[pl.ds(4 * gr, 4)] = ag16[s].astype(F32)


def kernel(x, Wq, Wo, K_ext, V_ext):
    my = lax.axis_index("i")
    K_loc = lax.dynamic_slice_in_dim(K_ext, my * KVL, KVL, axis=2)
    V_loc = lax.dynamic_slice_in_dim(V_ext, my * KVL, KVL, axis=2)
    K_loc = K_loc.reshape(B, Skv, KVL * Dh)
    V_loc = V_loc.reshape(B, Skv, KVL * Dh)
    x2 = x.reshape(R, D)

    out = pl.pallas_call(
        _body,
        out_shape=jax.ShapeDtypeStruct((N_DEV, CH, D), jnp.float32),
        in_specs=[pl.BlockSpec(memory_space=pltpu.VMEM)] * 5,
        out_specs=pl.BlockSpec(memory_space=pltpu.VMEM),
        scratch_shapes=[
            pltpu.VMEM((4, CH, D), BF),
            pltpu.VMEM((3, 4, CH, D), BF),
            pltpu.VMEM((2, CH, D), BF),
            pltpu.VMEM((1, CH, D), BF),
            pltpu.VMEM((1, CH, D), BF),
            pltpu.VMEM((2, CH, D), BF),
            pltpu.VMEM((3, 4, CH, D), BF),
            pltpu.SemaphoreType.DMA,
            pltpu.SemaphoreType.DMA((3,)),
            pltpu.SemaphoreType.DMA,
            pltpu.SemaphoreType.DMA((4,)),
            pltpu.SemaphoreType.DMA,
            pltpu.SemaphoreType.DMA((3,)),
        ],
        compiler_params=pltpu.CompilerParams(collective_id=0),
    )(x2, Wq, Wo, K_loc, V_loc)
    return out.reshape(B, Sq, D)


# device time: 108251 ns/iter; 1.8455x vs baseline; 1.1222x over previous
import jax
import jax.numpy as jnp
from jax import lax
from jax.experimental import pallas as pl
from jax.experimental.pallas import tpu as pltpu

N_DEV = 16
B, Sq, D = 4, 256, 1024
HL, Dh = 8, 128
KVL = 2
GRP = 4
Skv = 1024
R = B * Sq
CH = R // N_DEV
SCALE = 0.08838834764831843
BF = jnp.bfloat16
F32 = jnp.float32


def _rdma(src, dst, send_sem, recv_sem, dev):
    return pltpu.make_async_remote_copy(
        src_ref=src, dst_ref=dst, send_sem=send_sem, recv_sem=recv_sem,
        device_id=(dev,), device_id_type=pl.DeviceIdType.MESH,
    )


def _body(x_ref, wq_ref, wo_ref, k_ref, v_ref, out_ref,
          st16, pst16, agst, rs16, zA16, zB16, gb16, ga16, ag16,
          p1_send_sems, p1_recv_sems,
          p2_send_sem, p2_recv_sems,
          p3_send_sems, p3_recv_sems):
    my = lax.axis_index("i")
    z = my // 4
    w = my % 4
    zb0 = z % 2
    zb1 = (z // 2) % 2
    wr = z * 4 + (w + 1) % 4
    wl = z * 4 + (w + 3) % 4
    pA = my + 4 - 8 * zb0
    pB = my + 8 - 16 * zb1

    barrier = pltpu.get_barrier_semaphore()
    for nbr in (wl, wr, pA, pB):
        pl.semaphore_signal(barrier, inc=1, device_id=(nbr,),
                            device_id_type=pl.DeviceIdType.MESH)
    pl.semaphore_wait(barrier, 4)

    def compute_group(g):
        xg = x_ref[pl.ds(Sq * g, Sq)]
        qg = jnp.dot(xg, wq_ref[...],
                     preferred_element_type=F32)
        kb = k_ref[pl.ds(g, 1)].reshape(Skv, KVL * Dh)
        vb = v_ref[pl.ds(g, 1)].reshape(Skv, KVL * Dh)
        heads = []
        for h in range(HL):
            kv = h // GRP
            qh = qg[:, h * Dh:(h + 1) * Dh]
            khd = kb[:, kv * Dh:(kv + 1) * Dh]
            s = lax.dot_general(
                qh, khd, (((1,), (1,)), ((), ())),
                preferred_element_type=F32) * SCALE
            m = jnp.max(s, axis=1, keepdims=True)
            p = jnp.exp(s - m)
            l = jnp.sum(p, axis=1, keepdims=True)
            o = jnp.dot(p, vb[:, kv * Dh:(kv + 1) * Dh],
                        preferred_element_type=F32)
            heads.append(o / l)
        rowb = jnp.concatenate(heads, axis=1)
        partial = jnp.dot(rowb, wo_ref[...],
                          preferred_element_type=F32)
        out_ref[pl.ds(4 * g, 4)] = partial.reshape(4, CH, D)

    compute_group(w)
    p1_descs = []
    for s in range(3):
        g = (w - s + 4) % 4
        st16[s] = out_ref[pl.ds(4 * g, 4)].astype(BF)
        r = _rdma(st16.at[s], rs16.at[s],
                  p1_send_sems.at[s], p1_recv_sems.at[s], wr)
        r.start()
        p1_descs.append(r)
        nxt = (w - 1 - s + 4) % 4
        compute_group(nxt)
        r.wait_recv()
        out_ref[pl.ds(4 * nxt, 4)] = (out_ref[pl.ds(4 * nxt, 4)]
                                      + rs16[s].astype(F32))
    for r in p1_descs:
        r.wait_send()
    G = (w + 1) % 4
    base = 4 * G
    kA = base + 2 * zb0

    sA = base + 2 * (1 - zb0)
    pst16[pl.ds(0, 2)] = out_ref[pl.ds(sA, 2)].astype(BF)
    r = _rdma(pst16.at[pl.ds(0, 2)], zA16,
              p2_send_sem, p2_recv_sems.at[0], pA)
    r.start()
    r.wait()
    out_ref[pl.ds(kA, 2)] = out_ref[pl.ds(kA, 2)] + zA16[...].astype(F32)

    kb_ = kA + zb1
    sb_ = kA + (1 - zb1)
    pst16[pl.ds(0, 1)] = out_ref[pl.ds(sb_, 1)].astype(BF)
    r = _rdma(pst16.at[pl.ds(0, 1)], zB16,
              p2_send_sem, p2_recv_sems.at[1], pB)
    r.start()
    r.wait()
    out_ref[pl.ds(kb_, 1)] = out_ref[pl.ds(kb_, 1)] + zB16[...].astype(F32)

    pst16[pl.ds(0, 1)] = out_ref[pl.ds(kb_, 1)].astype(BF)
    r = _rdma(pst16.at[pl.ds(0, 1)], gb16,
              p2_send_sem, p2_recv_sems.at[2], pB)
    r.start()
    r.wait()
    out_ref[pl.ds(sb_, 1)] = gb16[...].astype(F32)

    pst16[pl.ds(zb1, 1)] = out_ref[pl.ds(kb_, 1)].astype(BF)
    pst16[pl.ds(1 - zb1, 1)] = gb16[...]
    r = _rdma(pst16.at[pl.ds(0, 2)], ga16,
              p2_send_sem, p2_recv_sems.at[3], pA)
    r.start()
    r.wait()
    oth = base + 2 * (1 - zb0)
    out_ref[pl.ds(oth, 2)] = ga16[...].astype(F32)

    agst[pl.ds(2 * zb0 + zb1, 1)] = out_ref[pl.ds(kb_, 1)].astype(BF)
    agst[pl.ds(2 * zb0 + 1 - zb1, 1)] = gb16[...]
    agst[pl.ds(2 * (1 - zb0), 2)] = ga16[...]
    p3_descs = []
    for s in range(3):
        src = agst if s == 0 else ag16.at[s - 1]
        r = _rdma(src, ag16.at[s], p3_send_sems.at[s], p3_recv_sems.at[s], wr)
        r.start()
        p3_descs.append(r)
        if s > 0:
            gr = (w - (s - 1) + 4) % 4
            out_ref[pl.ds(4 * gr, 4)] = ag16[s - 1].astype(F32)
        r.wait_recv()
    out_ref[pl.ds(4 * ((w - 2 + 4) % 4), 4)] = ag16[2].astype(F32)
    for r in p3_descs:
        r.wait_send()


def kernel(x, Wq, Wo, K_ext, V_ext):
    my = lax.axis_index("i")
    K_loc = lax.dynamic_slice_in_dim(K_ext, my * KVL, KVL, axis=2)
    V_loc = lax.dynamic_slice_in_dim(V_ext, my * KVL, KVL, axis=2)
    K_loc = K_loc.reshape(B, Skv, KVL * Dh)
    V_loc = V_loc.reshape(B, Skv, KVL * Dh)
    x2 = x.reshape(R, D)

    out = pl.pallas_call(
        _body,
        out_shape=jax.ShapeDtypeStruct((N_DEV, CH, D), jnp.float32),
        in_specs=[pl.BlockSpec(memory_space=pltpu.VMEM)] * 5,
        out_specs=pl.BlockSpec(memory_space=pltpu.VMEM),
        scratch_shapes=[
            pltpu.VMEM((3, 4, CH, D), BF),
            pltpu.VMEM((2, CH, D), BF),
            pltpu.VMEM((4, CH, D), BF),
            pltpu.VMEM((3, 4, CH, D), BF),
            pltpu.VMEM((2, CH, D), BF),
            pltpu.VMEM((1, CH, D), BF),
            pltpu.VMEM((1, CH, D), BF),
            pltpu.VMEM((2, CH, D), BF),
            pltpu.VMEM((3, 4, CH, D), BF),
            pltpu.SemaphoreType.DMA((3,)),
            pltpu.SemaphoreType.DMA((3,)),
            pltpu.SemaphoreType.DMA,
            pltpu.SemaphoreType.DMA((4,)),
            pltpu.SemaphoreType.DMA((3,)),
            pltpu.SemaphoreType.DMA((3,)),
        ],
        compiler_params=pltpu.CompilerParams(collective_id=0),
    )(x2, Wq, Wo, K_loc, V_loc)
    return out.reshape(B, Sq, D)


# device time: 107956 ns/iter; 1.8505x vs baseline; 1.0027x over previous
import jax
import jax.numpy as jnp
from jax import lax
from jax.experimental import pallas as pl
from jax.experimental.pallas import tpu as pltpu

N_DEV = 16
B, Sq, D = 4, 256, 1024
HL, Dh = 8, 128
KVL = 2
GRP = 4
Skv = 1024
R = B * Sq
CH = R // N_DEV
SCALE = 0.08838834764831843
BF = jnp.bfloat16
F32 = jnp.float32


def _rdma(src, dst, send_sem, recv_sem, dev):
    return pltpu.make_async_remote_copy(
        src_ref=src, dst_ref=dst, send_sem=send_sem, recv_sem=recv_sem,
        device_id=(dev,), device_id_type=pl.DeviceIdType.MESH,
    )


def _body(x_ref, wq_ref, wo_ref, k_ref, v_ref, out_ref,
          st16, pst16, agst, rs16, zA16, zB16, gb16, ga16, ag16,
          p1_send_sems, p1_recv_sems,
          p2_send_sem, p2_recv_sems,
          p3_send_sems, p3_recv_sems):
    my = lax.axis_index("i")
    z = my // 4
    w = my % 4
    zb0 = z % 2
    zb1 = (z // 2) % 2
    wr = z * 4 + (w + 1) % 4
    wl = z * 4 + (w + 3) % 4
    pA = my + 4 - 8 * zb0
    pB = my + 8 - 16 * zb1

    barrier = pltpu.get_barrier_semaphore()
    for nbr in (wl, wr, pA, pB):
        pl.semaphore_signal(barrier, inc=1, device_id=(nbr,),
                            device_id_type=pl.DeviceIdType.MESH)
    pl.semaphore_wait(barrier, 4)

    def compute_group(g):
        xg = x_ref[pl.ds(Sq * g, Sq)]
        qg = jnp.dot(xg, wq_ref[...],
                     preferred_element_type=F32)
        kb = k_ref[pl.ds(g, 1)].reshape(Skv, KVL * Dh)
        vb = v_ref[pl.ds(g, 1)].reshape(Skv, KVL * Dh)
        heads = []
        for h in range(HL):
            kv = h // GRP
            qh = qg[:, h * Dh:(h + 1) * Dh].astype(BF)
            khd = kb[:, kv * Dh:(kv + 1) * Dh]
            s = lax.dot_general(
                qh, khd, (((1,), (1,)), ((), ())),
                preferred_element_type=F32) * SCALE
            m = jnp.max(s, axis=1, keepdims=True)
            p = jnp.exp(s - m)
            l = jnp.sum(p, axis=1, keepdims=True)
            o = jnp.dot(p.astype(BF), vb[:, kv * Dh:(kv + 1) * Dh],
                        preferred_element_type=F32)
            heads.append(o / l)
        rowb = jnp.concatenate(heads, axis=1).astype(BF)
        partial = jnp.dot(rowb, wo_ref[...],
                          preferred_element_type=F32)
        out_ref[pl.ds(4 * g, 4)] = partial.reshape(4, CH, D)

    compute_group(w)
    p1_descs = []
    for s in range(3):
        g = (w - s + 4) % 4
        st16[s] = out_ref[pl.ds(4 * g, 4)].astype(BF)
        r = _rdma(st16.at[s], rs16.at[s],
                  p1_send_sems.at[s], p1_recv_sems.at[s], wr)
        r.start()
        p1_descs.append(r)
        nxt = (w - 1 - s + 4) % 4
        compute_group(nxt)
        r.wait_recv()
        out_ref[pl.ds(4 * nxt, 4)] = (out_ref[pl.ds(4 * nxt, 4)]
                                      + rs16[s].astype(F32))
    for r in p1_descs:
        r.wait_send()
    G = (w + 1) % 4
    base = 4 * G
    kA = base + 2 * zb0

    sA = base + 2 * (1 - zb0)
    pst16[pl.ds(0, 2)] = out_ref[pl.ds(sA, 2)].astype(BF)
    r = _rdma(pst16.at[pl.ds(0, 2)], zA16,
              p2_send_sem, p2_recv_sems.at[0], pA)
    r.start()
    r.wait()
    out_ref[pl.ds(kA, 2)] = out_ref[pl.ds(kA, 2)] + zA16[...].astype(F32)

    kb_ = kA + zb1
    sb_ = kA + (1 - zb1)
    pst16[pl.ds(0, 1)] = out_ref[pl.ds(sb_, 1)].astype(BF)
    r = _rdma(pst16.at[pl.ds(0, 1)], zB16,
              p2_send_sem, p2_recv_sems.at[1], pB)
    r.start()
    r.wait()
    out_ref[pl.ds(kb_, 1)] = out_ref[pl.ds(kb_, 1)] + zB16[...].astype(F32)

    pst16[pl.ds(0, 1)] = out_ref[pl.ds(kb_, 1)].astype(BF)
    r = _rdma(pst16.at[pl.ds(0, 1)], gb16,
              p2_send_sem, p2_recv_sems.at[2], pB)
    r.start()
    r.wait()
    out_ref[pl.ds(sb_, 1)] = gb16[...].astype(F32)

    pst16[pl.ds(zb1, 1)] = out_ref[pl.ds(kb_, 1)].astype(BF)
    pst16[pl.ds(1 - zb1, 1)] = gb16[...]
    r = _rdma(pst16.at[pl.ds(0, 2)], ga16,
              p2_send_sem, p2_recv_sems.at[3], pA)
    r.start()
    r.wait()
    oth = base + 2 * (1 - zb0)
    out_ref[pl.ds(oth, 2)] = ga16[...].astype(F32)

    agst[pl.ds(2 * zb0 + zb1, 1)] = out_ref[pl.ds(kb_, 1)].astype(BF)
    agst[pl.ds(2 * zb0 + 1 - zb1, 1)] = gb16[...]
    agst[pl.ds(2 * (1 - zb0), 2)] = ga16[...]
    p3_descs = []
    for s in range(3):
        src = agst if s == 0 else ag16.at[s - 1]
        r = _rdma(src, ag16.at[s], p3_send_sems.at[s], p3_recv_sems.at[s], wr)
        r.start()
        p3_descs.append(r)
        if s > 0:
            gr = (w - (s - 1) + 4) % 4
            out_ref[pl.ds(4 * gr, 4)] = ag16[s - 1].astype(F32)
        r.wait_recv()
    out_ref[pl.ds(4 * ((w - 2 + 4) % 4), 4)] = ag16[2].astype(F32)
    for r in p3_descs:
        r.wait_send()


def kernel(x, Wq, Wo, K_ext, V_ext):
    my = lax.axis_index("i")
    K_loc = lax.dynamic_slice_in_dim(K_ext, my * KVL, KVL, axis=2)
    V_loc = lax.dynamic_slice_in_dim(V_ext, my * KVL, KVL, axis=2)
    K_loc = K_loc.reshape(B, Skv, KVL * Dh).astype(BF)
    V_loc = V_loc.reshape(B, Skv, KVL * Dh).astype(BF)
    x2 = x.reshape(R, D).astype(BF)
    Wq16 = Wq.astype(BF)
    Wo16 = Wo.astype(BF)

    out = pl.pallas_call(
        _body,
        out_shape=jax.ShapeDtypeStruct((N_DEV, CH, D), jnp.float32),
        in_specs=[pl.BlockSpec(memory_space=pltpu.VMEM)] * 5,
        out_specs=pl.BlockSpec(memory_space=pltpu.VMEM),
        scratch_shapes=[
            pltpu.VMEM((3, 4, CH, D), BF),
            pltpu.VMEM((2, CH, D), BF),
            pltpu.VMEM((4, CH, D), BF),
            pltpu.VMEM((3, 4, CH, D), BF),
            pltpu.VMEM((2, CH, D), BF),
            pltpu.VMEM((1, CH, D), BF),
            pltpu.VMEM((1, CH, D), BF),
            pltpu.VMEM((2, CH, D), BF),
            pltpu.VMEM((3, 4, CH, D), BF),
            pltpu.SemaphoreType.DMA((3,)),
            pltpu.SemaphoreType.DMA((3,)),
            pltpu.SemaphoreType.DMA,
            pltpu.SemaphoreType.DMA((4,)),
            pltpu.SemaphoreType.DMA((3,)),
            pltpu.SemaphoreType.DMA((3,)),
        ],
        compiler_params=pltpu.CompilerParams(collective_id=0),
    )(x2, Wq16, Wo16, K_loc, V_loc)
    return out.reshape(B, Sq, D)


# device time: 102870 ns/iter; 1.9420x vs baseline; 1.0494x over previous
import jax
import jax.numpy as jnp
from jax import lax
from jax.experimental import pallas as pl
from jax.experimental.pallas import tpu as pltpu

N_DEV = 16
B, Sq, D = 4, 256, 1024
HL, Dh = 8, 128
KVL = 2
GRP = 4
Skv = 1024
R = B * Sq
CH = R // N_DEV
SCALE = 0.08838834764831843
BF = jnp.bfloat16
F32 = jnp.float32


def _rdma(src, dst, send_sem, recv_sem, dev):
    return pltpu.make_async_remote_copy(
        src_ref=src, dst_ref=dst, send_sem=send_sem, recv_sem=recv_sem,
        device_id=(dev,), device_id_type=pl.DeviceIdType.MESH,
    )


def _body(x_ref, wq_ref, wo_ref, k_ref, v_ref, out_ref,
          st16, pst16, rs16, zA16, zB16, agstR, agstL, agR, agL,
          p1_send_sems, p1_recv_sems,
          p2_send_sem, p2_recv_sems,
          p3r_send_sems, p3r_recv_sems,
          p3l_send_sems, p3l_recv_sems):
    my = lax.axis_index("i")
    z = my // 4
    w = my % 4
    zb0 = z % 2
    zb1 = (z // 2) % 2
    wr = z * 4 + (w + 1) % 4
    wl = z * 4 + (w + 3) % 4
    pA = my + 4 - 8 * zb0
    pB = my + 8 - 16 * zb1

    barrier = pltpu.get_barrier_semaphore()
    for nbr in (wl, wr, pA, pB):
        pl.semaphore_signal(barrier, inc=1, device_id=(nbr,),
                            device_id_type=pl.DeviceIdType.MESH)
    pl.semaphore_wait(barrier, 4)

    def compute_group(g):
        xg = x_ref[pl.ds(Sq * g, Sq)]
        qg = jnp.dot(xg, wq_ref[...],
                     preferred_element_type=F32)
        kb = k_ref[pl.ds(g, 1)].reshape(Skv, KVL * Dh)
        vb = v_ref[pl.ds(g, 1)].reshape(Skv, KVL * Dh)
        heads = []
        for h in range(HL):
            kv = h // GRP
            qh = qg[:, h * Dh:(h + 1) * Dh].astype(BF)
            khd = kb[:, kv * Dh:(kv + 1) * Dh]
            s = lax.dot_general(
                qh, khd, (((1,), (1,)), ((), ())),
                preferred_element_type=F32) * SCALE
            m = jnp.max(s, axis=1, keepdims=True)
            p = jnp.exp(s - m)
            l = jnp.sum(p, axis=1, keepdims=True)
            o = jnp.dot(p.astype(BF), vb[:, kv * Dh:(kv + 1) * Dh],
                        preferred_element_type=F32)
            heads.append(o / l)
        rowb = jnp.concatenate(heads, axis=1).astype(BF)
        partial = jnp.dot(rowb, wo_ref[...],
                          preferred_element_type=F32)
        out_ref[pl.ds(4 * g, 4)] = partial.reshape(4, CH, D)

    compute_group(w)
    p1_descs = []
    for s in range(3):
        g = (w - s + 4) % 4
        st16[s] = out_ref[pl.ds(4 * g, 4)].astype(BF)
        r = _rdma(st16.at[s], rs16.at[s],
                  p1_send_sems.at[s], p1_recv_sems.at[s], wr)
        r.start()
        p1_descs.append(r)
        nxt = (w - 1 - s + 4) % 4
        compute_group(nxt)
        r.wait_recv()
        out_ref[pl.ds(4 * nxt, 4)] = (out_ref[pl.ds(4 * nxt, 4)]
                                      + rs16[s].astype(F32))
    for r in p1_descs:
        r.wait_send()
    G = (w + 1) % 4
    base = 4 * G

    pst16[...] = out_ref[pl.ds(base, 4)].astype(BF)
    r = _rdma(pst16, zA16, p2_send_sem, p2_recv_sems.at[0], pA)
    r.start()
    r.wait()
    out_ref[pl.ds(base, 4)] = (out_ref[pl.ds(base, 4)]
                               + zA16[...].astype(F32))
    pst16[...] = out_ref[pl.ds(base, 4)].astype(BF)
    r = _rdma(pst16, zB16, p2_send_sem, p2_recv_sems.at[1], pB)
    r.start()
    r.wait()
    out_ref[pl.ds(base, 4)] = (out_ref[pl.ds(base, 4)]
                               + zB16[...].astype(F32))

    agstR[...] = out_ref[pl.ds(base, 2)].astype(BF)
    agstL[...] = out_ref[pl.ds(base + 2, 2)].astype(BF)
    p3_descs = []
    for s in range(3):
        srcR = agstR if s == 0 else agR.at[s - 1]
        rR = _rdma(srcR, agR.at[s],
                   p3r_send_sems.at[s], p3r_recv_sems.at[s], wr)
        rR.start()
        srcL = agstL if s == 0 else agL.at[s - 1]
        rL = _rdma(srcL, agL.at[s],
                   p3l_send_sems.at[s], p3l_recv_sems.at[s], wl)
        rL.start()
        p3_descs += [rR, rL]
        if s > 0:
            gr = (w - (s - 1) + 4) % 4
            out_ref[pl.ds(4 * gr, 2)] = agR[s - 1].astype(F32)
            gl = (w + 2 + (s - 1)) % 4
            out_ref[pl.ds(4 * gl + 2, 2)] = agL[s - 1].astype(F32)
        rR.wait_recv()
        rL.wait_recv()
    out_ref[pl.ds(4 * ((w - 2 + 4) % 4), 2)] = agR[2].astype(F32)
    out_ref[pl.ds(4 * (w % 4) + 2, 2)] = agL[2].astype(F32)
    for r in p3_descs:
        r.wait_send()


def kernel(x, Wq, Wo, K_ext, V_ext):
    my = lax.axis_index("i")
    K_loc = lax.dynamic_slice_in_dim(K_ext, my * KVL, KVL, axis=2)
    V_loc = lax.dynamic_slice_in_dim(V_ext, my * KVL, KVL, axis=2)
    K_loc = K_loc.reshape(B, Skv, KVL * Dh).astype(BF)
    V_loc = V_loc.reshape(B, Skv, KVL * Dh).astype(BF)
    x2 = x.reshape(R, D).astype(BF)
    Wq16 = Wq.astype(BF)
    Wo16 = Wo.astype(BF)

    out = pl.pallas_call(
        _body,
        out_shape=jax.ShapeDtypeStruct((N_DEV, CH, D), jnp.float32),
        in_specs=[pl.BlockSpec(memory_space=pltpu.VMEM)] * 5,
        out_specs=pl.BlockSpec(memory_space=pltpu.VMEM),
        scratch_shapes=[
            pltpu.VMEM((3, 4, CH, D), BF),
            pltpu.VMEM((4, CH, D), BF),
            pltpu.VMEM((3, 4, CH, D), BF),
            pltpu.VMEM((4, CH, D), BF),
            pltpu.VMEM((4, CH, D), BF),
            pltpu.VMEM((2, CH, D), BF),
            pltpu.VMEM((2, CH, D), BF),
            pltpu.VMEM((3, 2, CH, D), BF),
            pltpu.VMEM((3, 2, CH, D), BF),
            pltpu.SemaphoreType.DMA((3,)),
            pltpu.SemaphoreType.DMA((3,)),
            pltpu.SemaphoreType.DMA,
            pltpu.SemaphoreType.DMA((2,)),
            pltpu.SemaphoreType.DMA((3,)),
            pltpu.SemaphoreType.DMA((3,)),
            pltpu.SemaphoreType.DMA((3,)),
            pltpu.SemaphoreType.DMA((3,)),
        ],
        compiler_params=pltpu.CompilerParams(collective_id=0),
    )(x2, Wq16, Wo16, K_loc, V_loc)
    return out.reshape(B, Sq, D)
